# baseline (speedup 1.0000x reference)
import functools

import jax
import jax.numpy as jnp
from jax import lax
from jax.experimental import pallas as pl
from jax.experimental.pallas import tpu as pltpu
from jax.experimental.pallas import tpu_sc as plsc

NUM_CORES = 2
NUM_SUBCORES = 16
LANES = 16
NW = NUM_CORES * NUM_SUBCORES

NUM_ENTITIES = 1000000
EMBED_DIM = 64
D2 = 2 * EMBED_DIM
BATCH = 16384
B_PER_W = BATCH // NW

BIGLANES = 384
BIGROWS = BIGLANES // 2
NBIG = NUM_ENTITIES // BIGLANES
BIG_PER_W = 81
NREM = NBIG - NW * BIG_PER_W
SCRATCH_ROWS = 500032

CHUNK = 64
NCHUNK = B_PER_W // CHUNK
GROUPS = CHUNK // LANES


def _transpose_block(inbuf, outbuf, ngroups):
    iota = lax.iota(jnp.int32, LANES)
    row_pat = lax.shift_right_logical(iota, 1)
    col_pat = (iota & 1) * EMBED_DIM

    def grp(g, _):
        e0 = g * LANES
        rbase = row_pat + g * (LANES // 2)
        for d0 in range(0, EMBED_DIM, 8):
            vals = [inbuf[d0 + i, pl.ds(e0, LANES)] for i in range(8)]
            for i, v in enumerate(vals):
                plsc.store_scatter(outbuf, [rbase, col_pat + (d0 + i)], v)
        return 0

    lax.fori_loop(0, ngroups, grp, 0)


def _fmt_kernel(et_hbm, out_hbm, in0, in1, out0, out1,
                sem_i0, sem_i1, sem_o0, sem_o1):
    wid = lax.axis_index("s") * NUM_CORES + lax.axis_index("c")
    start = wid * BIG_PER_W

    def fire_in(b, buf, sem):
        lane = pl.multiple_of(b * BIGLANES, 128)
        for dblk in range(EMBED_DIM // 8):
            cp = pltpu.async_copy(
                et_hbm.at[pl.ds(dblk * 8, 8), pl.ds(lane, BIGLANES)],
                buf.at[pl.ds(dblk * 8, 8), :], sem)
        return cp

    def fire_out(b, buf, sem):
        row = pl.multiple_of(b * BIGROWS, 8)
        return pltpu.async_copy(buf, out_hbm.at[pl.ds(row, BIGROWS), :], sem)

    def wait_in(buf, sem):
        for dblk in range(EMBED_DIM // 8):
            pltpu.make_async_copy(
                et_hbm.at[pl.ds(0, 8), pl.ds(0, BIGLANES)],
                buf.at[pl.ds(dblk * 8, 8), :], sem).wait()

    def wait_out(buf, sem):
        pltpu.make_async_copy(
            buf, out_hbm.at[pl.ds(0, BIGROWS), :], sem).wait()

    fire_in(start, in0, sem_i0)
    fire_in(start + 1, in1, sem_i1)

    def body(k, _):
        b = start + 2 * k
        for (bb, ibuf, isem, obuf, osem) in (
                (b, in0, sem_i0, out0, sem_o0),
                (b + 1, in1, sem_i1, out1, sem_o1)):
            wait_in(ibuf, isem)

            @pl.when(k > 0)
            def _():
                wait_out(obuf, osem)

            _transpose_block(ibuf, obuf, BIGLANES // LANES)
            fire_out(bb, obuf, osem)
            fire_in(bb + 2, ibuf, isem)
        return 0

    lax.fori_loop(0, (BIG_PER_W - 1) // 2, body, 0)

    wait_in(in0, sem_i0)
    wait_out(out0, sem_o0)
    _transpose_block(in0, out0, BIGLANES // LANES)
    fire_out(start + BIG_PER_W - 1, out0, sem_o0)
    wait_in(in1, sem_i1)
    wait_out(out1, sem_o1)
    wait_out(out0, sem_o0)

    @pl.when(wid < NREM)
    def _():
        bb = NW * BIG_PER_W + wid
        fire_in(bb, in0, sem_i0)
        wait_in(in0, sem_i0)
        _transpose_block(in0, out0, BIGLANES // LANES)
        fire_out(bb, out0, sem_o0).wait()

    @pl.when(wid == NREM)
    def _():
        lane = pl.multiple_of(NBIG * BIGLANES, 128)
        pltpu.async_copy(
            et_hbm.at[:, pl.ds(lane, 128)],
            in0.at[:, pl.ds(0, 128)], sem_i0).wait()
        _transpose_block(in0, out0, 128 // LANES)
        pltpu.async_copy(
            out0.at[pl.ds(0, 64), :],
            out_hbm.at[pl.ds(pl.multiple_of(NBIG * BIGROWS, 8), 64), :],
            sem_o0).wait()


def _score_kernel(s_hbm, p_hbm, o_hbm, e2_hbm, r2_hbm, out_hbm,
                  s2, p2, o2, hs, hp, ho,
                  sbuf, pbuf, obuf, stage, out_v,
                  sem_s, sem_p, sem_o):
    wid = lax.axis_index("s") * NUM_CORES + lax.axis_index("c")
    base = wid * B_PER_W

    pltpu.sync_copy(s_hbm.at[pl.ds(base, B_PER_W)], s2)
    pltpu.sync_copy(p_hbm.at[pl.ds(base, B_PER_W)], p2)
    pltpu.sync_copy(o_hbm.at[pl.ds(base, B_PER_W)], o2)
    for g in range(B_PER_W // LANES):
        sl = pl.ds(g * LANES, LANES)
        for idx_ref, h_ref in ((s2, hs), (p2, hp), (o2, ho)):
            v = idx_ref[sl]
            h_ref[sl] = (v & 1) * EMBED_DIM
            idx_ref[sl] = v >> 1

    def chunk_body(c, _):
        cb = c * CHUNK
        cp_s = pltpu.async_copy(e2_hbm.at[s2.at[pl.ds(cb, CHUNK)]], sbuf, sem_s)
        cp_p = pltpu.async_copy(r2_hbm.at[p2.at[pl.ds(cb, CHUNK)]], pbuf, sem_p)
        cp_o = pltpu.async_copy(e2_hbm.at[o2.at[pl.ds(cb, CHUNK)]], obuf, sem_o)
        cp_s.wait()
        cp_p.wait()
        cp_o.wait()

        for g in range(GROUPS):
            gsl = pl.ds(cb + g * LANES, LANES)
            hsv = hs[gsl]
            hpv = hp[gsl]
            hov = ho[gsl]
            for j in range(LANES):
                row = g * LANES + j
                a = hsv[j]
                b = hpv[j]
                cofs = hov[j]
                acc = (sbuf[row, pl.ds(a, LANES)]
                       * pbuf[row, pl.ds(b, LANES)]
                       * obuf[row, pl.ds(cofs, LANES)])
                for d in range(1, EMBED_DIM // LANES):
                    acc = acc + (sbuf[row, pl.ds(a + d * LANES, LANES)]
                                 * pbuf[row, pl.ds(b + d * LANES, LANES)]
                                 * obuf[row, pl.ds(cofs + d * LANES, LANES)])
                stage[pl.ds(j * LANES, LANES)] = acc
            col = lax.iota(jnp.int32, LANES) * LANES
            out_vec = plsc.load_gather(stage, [col])
            for l in range(1, LANES):
                out_vec = out_vec + plsc.load_gather(stage, [col + l])
            out_v[gsl] = out_vec
        return 0

    lax.fori_loop(0, NCHUNK, chunk_body, 0)

    pltpu.sync_copy(out_v, out_hbm.at[pl.ds(base, B_PER_W)])


_SC_PARAMS = pltpu.CompilerParams(
    needs_layout_passes=False, use_tc_tiling_on_sc=True)


@jax.jit
def _run(s, p, o, entity_embeddings, relation_embeddings):
    mesh = plsc.VectorSubcoreMesh(core_axis_name="c", subcore_axis_name="s")
    et = entity_embeddings.T
    fmt = functools.partial(
        pl.kernel,
        out_type=jax.ShapeDtypeStruct((SCRATCH_ROWS, D2), jnp.float32),
        mesh=mesh,
        compiler_params=_SC_PARAMS,
        scratch_types=[
            pltpu.VMEM((EMBED_DIM, BIGLANES), jnp.float32),
            pltpu.VMEM((EMBED_DIM, BIGLANES), jnp.float32),
            pltpu.VMEM((BIGROWS, D2), jnp.float32),
            pltpu.VMEM((BIGROWS, D2), jnp.float32),
            pltpu.SemaphoreType.DMA,
            pltpu.SemaphoreType.DMA,
            pltpu.SemaphoreType.DMA,
            pltpu.SemaphoreType.DMA,
        ],
    )(_fmt_kernel)
    e2 = fmt(et)

    r2 = relation_embeddings.reshape(
        relation_embeddings.shape[0] // 2, D2)
    score = functools.partial(
        pl.kernel,
        out_type=jax.ShapeDtypeStruct((BATCH,), jnp.float32),
        mesh=mesh,
        compiler_params=_SC_PARAMS,
        scratch_types=[
            pltpu.VMEM((B_PER_W,), jnp.int32),
            pltpu.VMEM((B_PER_W,), jnp.int32),
            pltpu.VMEM((B_PER_W,), jnp.int32),
            pltpu.VMEM((B_PER_W,), jnp.int32),
            pltpu.VMEM((B_PER_W,), jnp.int32),
            pltpu.VMEM((B_PER_W,), jnp.int32),
            pltpu.VMEM((CHUNK, D2), jnp.float32),
            pltpu.VMEM((CHUNK, D2), jnp.float32),
            pltpu.VMEM((CHUNK, D2), jnp.float32),
            pltpu.VMEM((LANES * LANES,), jnp.float32),
            pltpu.VMEM((B_PER_W,), jnp.float32),
            pltpu.SemaphoreType.DMA,
            pltpu.SemaphoreType.DMA,
            pltpu.SemaphoreType.DMA,
        ],
    )(_score_kernel)
    return score(s, p, o, e2, r2)


def kernel(s, p, o, entity_embeddings, relation_embeddings):
    return _run(s.astype(jnp.int32), p.astype(jnp.int32), o.astype(jnp.int32),
                entity_embeddings, relation_embeddings)


# speedup vs baseline: 1.0021x; 1.0021x over previous
import functools

import jax
import jax.numpy as jnp
from jax import lax
from jax.experimental import pallas as pl
from jax.experimental.pallas import tpu as pltpu
from jax.experimental.pallas import tpu_sc as plsc

NUM_CORES = 2
NUM_SUBCORES = 16
LANES = 16
NW = NUM_CORES * NUM_SUBCORES

NUM_ENTITIES = 1000000
EMBED_DIM = 64
D2 = 2 * EMBED_DIM
BATCH = 16384
B_PER_W = BATCH // NW

BIGLANES = 128
BIGROWS = BIGLANES // 2
NBIG = NUM_ENTITIES // BIGLANES
BIG_PER_W = 244
NREM = NBIG - NW * BIG_PER_W
NSLOT = 4
SCRATCH_ROWS = 500032

CHUNK = 64
NCHUNK = B_PER_W // CHUNK
GROUPS = CHUNK // LANES


def _transpose_block(inbuf, outbuf, ngroups):
    iota = lax.iota(jnp.int32, LANES)
    row_pat = lax.shift_right_logical(iota, 1)
    col_pat = (iota & 1) * EMBED_DIM

    def grp(g, _):
        e0 = g * LANES
        rbase = row_pat + g * (LANES // 2)
        for d0 in range(0, EMBED_DIM, 8):
            vals = [inbuf[d0 + i, pl.ds(e0, LANES)] for i in range(8)]
            for i, v in enumerate(vals):
                plsc.store_scatter(outbuf, [rbase, col_pat + (d0 + i)], v)
        return 0

    lax.fori_loop(0, ngroups, grp, 0)


def _fmt_kernel(et_hbm, out_hbm, in0, in1, in2, in3, out0, out1, out2, out3,
                sem_i0, sem_i1, sem_i2, sem_i3,
                sem_o0, sem_o1, sem_o2, sem_o3):
    wid = lax.axis_index("s") * NUM_CORES + lax.axis_index("c")
    start = wid * BIG_PER_W
    ins = ((in0, sem_i0), (in1, sem_i1), (in2, sem_i2), (in3, sem_i3))
    outs = ((out0, sem_o0), (out1, sem_o1), (out2, sem_o2), (out3, sem_o3))

    def fire_in(b, buf, sem):
        lane = pl.multiple_of(b * BIGLANES, 128)
        for dblk in range(EMBED_DIM // 8):
            cp = pltpu.async_copy(
                et_hbm.at[pl.ds(dblk * 8, 8), pl.ds(lane, BIGLANES)],
                buf.at[pl.ds(dblk * 8, 8), :], sem)
        return cp

    def fire_out(b, buf, sem):
        row = pl.multiple_of(b * BIGROWS, 8)
        return pltpu.async_copy(buf, out_hbm.at[pl.ds(row, BIGROWS), :], sem)

    def wait_in(buf, sem):
        for dblk in range(EMBED_DIM // 8):
            pltpu.make_async_copy(
                et_hbm.at[pl.ds(0, 8), pl.ds(0, BIGLANES)],
                buf.at[pl.ds(dblk * 8, 8), :], sem).wait()

    def wait_out(buf, sem):
        pltpu.make_async_copy(
            buf, out_hbm.at[pl.ds(0, BIGROWS), :], sem).wait()

    for sl in range(NSLOT):
        fire_in(start + sl, ins[sl][0], ins[sl][1])

    def body(k, _):
        b = start + NSLOT * k
        for sl in range(NSLOT):
            bb = b + sl
            ibuf, isem = ins[sl]
            obuf, osem = outs[sl]
            wait_in(ibuf, isem)

            @pl.when(k > 0)
            def _():
                wait_out(obuf, osem)

            _transpose_block(ibuf, obuf, BIGLANES // LANES)
            fire_out(bb, obuf, osem)
            fire_in(bb + NSLOT, ibuf, isem)
        return 0

    lax.fori_loop(0, BIG_PER_W // NSLOT, body, 0)

    for sl in range(NSLOT):
        wait_in(ins[sl][0], ins[sl][1])
        wait_out(outs[sl][0], outs[sl][1])

    @pl.when(wid < NREM)
    def _():
        bb = NW * BIG_PER_W + wid
        fire_in(bb, in0, sem_i0)
        wait_in(in0, sem_i0)
        _transpose_block(in0, out0, BIGLANES // LANES)
        fire_out(bb, out0, sem_o0).wait()

    @pl.when(wid == NREM)
    def _():
        lane = pl.multiple_of(NBIG * BIGLANES, 128)
        pltpu.async_copy(
            et_hbm.at[:, pl.ds(lane, 128)],
            in0.at[:, pl.ds(0, 128)], sem_i0).wait()
        _transpose_block(in0, out0, 128 // LANES)
        pltpu.async_copy(
            out0.at[pl.ds(0, 64), :],
            out_hbm.at[pl.ds(pl.multiple_of(NBIG * BIGROWS, 8), 64), :],
            sem_o0).wait()


def _score_kernel(s_hbm, p_hbm, o_hbm, e2_hbm, r2_hbm, out_hbm,
                  s2, p2, o2, hs, hp, ho,
                  sbuf, pbuf, obuf, stage, out_v,
                  sem_s, sem_p, sem_o):
    wid = lax.axis_index("s") * NUM_CORES + lax.axis_index("c")
    base = wid * B_PER_W

    pltpu.sync_copy(s_hbm.at[pl.ds(base, B_PER_W)], s2)
    pltpu.sync_copy(p_hbm.at[pl.ds(base, B_PER_W)], p2)
    pltpu.sync_copy(o_hbm.at[pl.ds(base, B_PER_W)], o2)
    for g in range(B_PER_W // LANES):
        sl = pl.ds(g * LANES, LANES)
        for idx_ref, h_ref in ((s2, hs), (p2, hp), (o2, ho)):
            v = idx_ref[sl]
            h_ref[sl] = (v & 1) * EMBED_DIM
            idx_ref[sl] = v >> 1

    def chunk_body(c, _):
        cb = c * CHUNK
        cp_s = pltpu.async_copy(e2_hbm.at[s2.at[pl.ds(cb, CHUNK)]], sbuf, sem_s)
        cp_p = pltpu.async_copy(r2_hbm.at[p2.at[pl.ds(cb, CHUNK)]], pbuf, sem_p)
        cp_o = pltpu.async_copy(e2_hbm.at[o2.at[pl.ds(cb, CHUNK)]], obuf, sem_o)
        cp_s.wait()
        cp_p.wait()
        cp_o.wait()

        for g in range(GROUPS):
            gsl = pl.ds(cb + g * LANES, LANES)
            hsv = hs[gsl]
            hpv = hp[gsl]
            hov = ho[gsl]
            for j in range(LANES):
                row = g * LANES + j
                a = hsv[j]
                b = hpv[j]
                cofs = hov[j]
                acc = (sbuf[row, pl.ds(a, LANES)]
                       * pbuf[row, pl.ds(b, LANES)]
                       * obuf[row, pl.ds(cofs, LANES)])
                for d in range(1, EMBED_DIM // LANES):
                    acc = acc + (sbuf[row, pl.ds(a + d * LANES, LANES)]
                                 * pbuf[row, pl.ds(b + d * LANES, LANES)]
                                 * obuf[row, pl.ds(cofs + d * LANES, LANES)])
                stage[pl.ds(j * LANES, LANES)] = acc
            col = lax.iota(jnp.int32, LANES) * LANES
            out_vec = plsc.load_gather(stage, [col])
            for l in range(1, LANES):
                out_vec = out_vec + plsc.load_gather(stage, [col + l])
            out_v[gsl] = out_vec
        return 0

    lax.fori_loop(0, NCHUNK, chunk_body, 0)

    pltpu.sync_copy(out_v, out_hbm.at[pl.ds(base, B_PER_W)])


_SC_PARAMS = pltpu.CompilerParams(
    needs_layout_passes=False, use_tc_tiling_on_sc=True)


@jax.jit
def _run(s, p, o, entity_embeddings, relation_embeddings):
    mesh = plsc.VectorSubcoreMesh(core_axis_name="c", subcore_axis_name="s")
    et = entity_embeddings.T
    fmt = functools.partial(
        pl.kernel,
        out_type=jax.ShapeDtypeStruct((SCRATCH_ROWS, D2), jnp.float32),
        mesh=mesh,
        compiler_params=_SC_PARAMS,
        scratch_types=(
            [pltpu.VMEM((EMBED_DIM, BIGLANES), jnp.float32)] * NSLOT
            + [pltpu.VMEM((BIGROWS, D2), jnp.float32)] * NSLOT
            + [pltpu.SemaphoreType.DMA] * (2 * NSLOT)
        ),
    )(_fmt_kernel)
    e2 = fmt(et)

    r2 = relation_embeddings.reshape(
        relation_embeddings.shape[0] // 2, D2)
    score = functools.partial(
        pl.kernel,
        out_type=jax.ShapeDtypeStruct((BATCH,), jnp.float32),
        mesh=mesh,
        compiler_params=_SC_PARAMS,
        scratch_types=[
            pltpu.VMEM((B_PER_W,), jnp.int32),
            pltpu.VMEM((B_PER_W,), jnp.int32),
            pltpu.VMEM((B_PER_W,), jnp.int32),
            pltpu.VMEM((B_PER_W,), jnp.int32),
            pltpu.VMEM((B_PER_W,), jnp.int32),
            pltpu.VMEM((B_PER_W,), jnp.int32),
            pltpu.VMEM((CHUNK, D2), jnp.float32),
            pltpu.VMEM((CHUNK, D2), jnp.float32),
            pltpu.VMEM((CHUNK, D2), jnp.float32),
            pltpu.VMEM((LANES * LANES,), jnp.float32),
            pltpu.VMEM((B_PER_W,), jnp.float32),
            pltpu.SemaphoreType.DMA,
            pltpu.SemaphoreType.DMA,
            pltpu.SemaphoreType.DMA,
        ],
    )(_score_kernel)
    return score(s, p, o, e2, r2)


def kernel(s, p, o, entity_embeddings, relation_embeddings):
    return _run(s.astype(jnp.int32), p.astype(jnp.int32), o.astype(jnp.int32),
                entity_embeddings, relation_embeddings)


# speedup vs baseline: 1.3573x; 1.3544x over previous
import functools

import jax
import jax.numpy as jnp
from jax import lax
from jax.experimental import pallas as pl
from jax.experimental.pallas import tpu as pltpu
from jax.experimental.pallas import tpu_sc as plsc

NUM_CORES = 2
NUM_SUBCORES = 16
LANES = 16
NW = NUM_CORES * NUM_SUBCORES

NUM_ENTITIES = 1000000
EMBED_DIM = 64
D2 = 2 * EMBED_DIM
BATCH = 16384
B_PER_W = BATCH // NW

BIGLANES = 128
BIGROWS = BIGLANES // 2
NBIG = NUM_ENTITIES // BIGLANES
BIG_PER_W = 244
NREM = NBIG - NW * BIG_PER_W
NSLOT = 4
SCRATCH_ROWS = 500032

CHUNK = 64
NCHUNK = B_PER_W // CHUNK
GROUPS = CHUNK // LANES


def _transpose_block(inbuf, outbuf, ngroups):
    iota = lax.iota(jnp.int32, LANES)
    wraps = [(iota + sh) & 15 for sh in range(LANES)]
    rows = [lax.shift_right_logical(w, 1) for w in wraps]
    cols = [(w & 1) * EMBED_DIM + iota for w in wraps]

    def grp(g, _):
        e0 = g * LANES
        r0 = g * (LANES // 2)
        for d0 in range(0, EMBED_DIM, LANES):
            for sh in range(LANES):
                v = plsc.load_gather(inbuf, [d0 + iota, e0 + wraps[sh]])
                plsc.store_scatter(outbuf, [r0 + rows[sh], d0 + cols[sh]], v)
        return 0

    lax.fori_loop(0, ngroups, grp, 0)


def _fmt_kernel(et_hbm, out_hbm, in0, in1, in2, in3, out0, out1, out2, out3,
                sem_i0, sem_i1, sem_i2, sem_i3,
                sem_o0, sem_o1, sem_o2, sem_o3):
    wid = lax.axis_index("s") * NUM_CORES + lax.axis_index("c")
    start = wid * BIG_PER_W
    ins = ((in0, sem_i0), (in1, sem_i1), (in2, sem_i2), (in3, sem_i3))
    outs = ((out0, sem_o0), (out1, sem_o1), (out2, sem_o2), (out3, sem_o3))

    def fire_in(b, buf, sem):
        lane = pl.multiple_of(b * BIGLANES, 128)
        for dblk in range(EMBED_DIM // 8):
            cp = pltpu.async_copy(
                et_hbm.at[pl.ds(dblk * 8, 8), pl.ds(lane, BIGLANES)],
                buf.at[pl.ds(dblk * 8, 8), :], sem)
        return cp

    def fire_out(b, buf, sem):
        row = pl.multiple_of(b * BIGROWS, 8)
        return pltpu.async_copy(buf, out_hbm.at[pl.ds(row, BIGROWS), :], sem)

    def wait_in(buf, sem):
        for dblk in range(EMBED_DIM // 8):
            pltpu.make_async_copy(
                et_hbm.at[pl.ds(0, 8), pl.ds(0, BIGLANES)],
                buf.at[pl.ds(dblk * 8, 8), :], sem).wait()

    def wait_out(buf, sem):
        pltpu.make_async_copy(
            buf, out_hbm.at[pl.ds(0, BIGROWS), :], sem).wait()

    for sl in range(NSLOT):
        fire_in(start + sl, ins[sl][0], ins[sl][1])

    def body(k, _):
        b = start + NSLOT * k
        for sl in range(NSLOT):
            bb = b + sl
            ibuf, isem = ins[sl]
            obuf, osem = outs[sl]
            wait_in(ibuf, isem)

            @pl.when(k > 0)
            def _():
                wait_out(obuf, osem)

            _transpose_block(ibuf, obuf, BIGLANES // LANES)
            fire_out(bb, obuf, osem)
            fire_in(bb + NSLOT, ibuf, isem)
        return 0

    lax.fori_loop(0, BIG_PER_W // NSLOT, body, 0)

    for sl in range(NSLOT):
        wait_in(ins[sl][0], ins[sl][1])
        wait_out(outs[sl][0], outs[sl][1])

    @pl.when(wid < NREM)
    def _():
        bb = NW * BIG_PER_W + wid
        fire_in(bb, in0, sem_i0)
        wait_in(in0, sem_i0)
        _transpose_block(in0, out0, BIGLANES // LANES)
        fire_out(bb, out0, sem_o0).wait()

    @pl.when(wid == NREM)
    def _():
        lane = pl.multiple_of(NBIG * BIGLANES, 128)
        pltpu.async_copy(
            et_hbm.at[:, pl.ds(lane, 128)],
            in0.at[:, pl.ds(0, 128)], sem_i0).wait()
        _transpose_block(in0, out0, 128 // LANES)
        pltpu.async_copy(
            out0.at[pl.ds(0, 64), :],
            out_hbm.at[pl.ds(pl.multiple_of(NBIG * BIGROWS, 8), 64), :],
            sem_o0).wait()


def _score_kernel(s_hbm, p_hbm, o_hbm, e2_hbm, r2_hbm, out_hbm,
                  s2, p2, o2, hs, hp, ho,
                  sbuf, pbuf, obuf, stage, out_v,
                  sem_s, sem_p, sem_o):
    wid = lax.axis_index("s") * NUM_CORES + lax.axis_index("c")
    base = wid * B_PER_W

    pltpu.sync_copy(s_hbm.at[pl.ds(base, B_PER_W)], s2)
    pltpu.sync_copy(p_hbm.at[pl.ds(base, B_PER_W)], p2)
    pltpu.sync_copy(o_hbm.at[pl.ds(base, B_PER_W)], o2)
    for g in range(B_PER_W // LANES):
        sl = pl.ds(g * LANES, LANES)
        for idx_ref, h_ref in ((s2, hs), (p2, hp), (o2, ho)):
            v = idx_ref[sl]
            h_ref[sl] = (v & 1) * EMBED_DIM
            idx_ref[sl] = v >> 1

    def chunk_body(c, _):
        cb = c * CHUNK
        cp_s = pltpu.async_copy(e2_hbm.at[s2.at[pl.ds(cb, CHUNK)]], sbuf, sem_s)
        cp_p = pltpu.async_copy(r2_hbm.at[p2.at[pl.ds(cb, CHUNK)]], pbuf, sem_p)
        cp_o = pltpu.async_copy(e2_hbm.at[o2.at[pl.ds(cb, CHUNK)]], obuf, sem_o)
        cp_s.wait()
        cp_p.wait()
        cp_o.wait()

        for g in range(GROUPS):
            gsl = pl.ds(cb + g * LANES, LANES)
            hsv = hs[gsl]
            hpv = hp[gsl]
            hov = ho[gsl]
            for j in range(LANES):
                row = g * LANES + j
                a = hsv[j]
                b = hpv[j]
                cofs = hov[j]
                acc = (sbuf[row, pl.ds(a, LANES)]
                       * pbuf[row, pl.ds(b, LANES)]
                       * obuf[row, pl.ds(cofs, LANES)])
                for d in range(1, EMBED_DIM // LANES):
                    acc = acc + (sbuf[row, pl.ds(a + d * LANES, LANES)]
                                 * pbuf[row, pl.ds(b + d * LANES, LANES)]
                                 * obuf[row, pl.ds(cofs + d * LANES, LANES)])
                stage[pl.ds(j * LANES, LANES)] = acc
            col = lax.iota(jnp.int32, LANES) * LANES
            out_vec = plsc.load_gather(stage, [col])
            for l in range(1, LANES):
                out_vec = out_vec + plsc.load_gather(stage, [col + l])
            out_v[gsl] = out_vec
        return 0

    lax.fori_loop(0, NCHUNK, chunk_body, 0)

    pltpu.sync_copy(out_v, out_hbm.at[pl.ds(base, B_PER_W)])


_SC_PARAMS = pltpu.CompilerParams(
    needs_layout_passes=False, use_tc_tiling_on_sc=True)


@jax.jit
def _run(s, p, o, entity_embeddings, relation_embeddings):
    mesh = plsc.VectorSubcoreMesh(core_axis_name="c", subcore_axis_name="s")
    et = entity_embeddings.T
    fmt = functools.partial(
        pl.kernel,
        out_type=jax.ShapeDtypeStruct((SCRATCH_ROWS, D2), jnp.float32),
        mesh=mesh,
        compiler_params=_SC_PARAMS,
        scratch_types=(
            [pltpu.VMEM((EMBED_DIM, BIGLANES), jnp.float32)] * NSLOT
            + [pltpu.VMEM((BIGROWS, D2), jnp.float32)] * NSLOT
            + [pltpu.SemaphoreType.DMA] * (2 * NSLOT)
        ),
    )(_fmt_kernel)
    e2 = fmt(et)

    r2 = relation_embeddings.reshape(
        relation_embeddings.shape[0] // 2, D2)
    score = functools.partial(
        pl.kernel,
        out_type=jax.ShapeDtypeStruct((BATCH,), jnp.float32),
        mesh=mesh,
        compiler_params=_SC_PARAMS,
        scratch_types=[
            pltpu.VMEM((B_PER_W,), jnp.int32),
            pltpu.VMEM((B_PER_W,), jnp.int32),
            pltpu.VMEM((B_PER_W,), jnp.int32),
            pltpu.VMEM((B_PER_W,), jnp.int32),
            pltpu.VMEM((B_PER_W,), jnp.int32),
            pltpu.VMEM((B_PER_W,), jnp.int32),
            pltpu.VMEM((CHUNK, D2), jnp.float32),
            pltpu.VMEM((CHUNK, D2), jnp.float32),
            pltpu.VMEM((CHUNK, D2), jnp.float32),
            pltpu.VMEM((LANES * LANES,), jnp.float32),
            pltpu.VMEM((B_PER_W,), jnp.float32),
            pltpu.SemaphoreType.DMA,
            pltpu.SemaphoreType.DMA,
            pltpu.SemaphoreType.DMA,
        ],
    )(_score_kernel)
    return score(s, p, o, e2, r2)


def kernel(s, p, o, entity_embeddings, relation_embeddings):
    return _run(s.astype(jnp.int32), p.astype(jnp.int32), o.astype(jnp.int32),
                entity_embeddings, relation_embeddings)


# speedup vs baseline: 2.9285x; 2.1575x over previous
import functools

import jax
import jax.numpy as jnp
from jax import lax
from jax.experimental import pallas as pl
from jax.experimental.pallas import tpu as pltpu
from jax.experimental.pallas import tpu_sc as plsc

NUM_CORES = 2
NUM_SUBCORES = 16
LANES = 16
NW = NUM_CORES * NUM_SUBCORES

NUM_ENTITIES = 1000000
EMBED_DIM = 64
D2 = 2 * EMBED_DIM
BATCH = 16384
B_PER_W = BATCH // NW

BIGLANES = 128
BIGROWS = BIGLANES // 2
NBIG = NUM_ENTITIES // BIGLANES
BIG_PER_W = 244
NREM = NBIG - NW * BIG_PER_W
NSLOT = 4
SCRATCH_ROWS = 500032

CHUNK = 64
NCHUNK = B_PER_W // CHUNK
GROUPS = CHUNK // LANES


def _transpose_block(inbuf, outbuf, ngroups):
    iota = lax.iota(jnp.int32, LANES)
    wraps = [(iota + sh) & 15 for sh in range(LANES)]
    rows = [lax.shift_right_logical(w, 1) for w in wraps]
    cols = [(w & 1) * EMBED_DIM + iota for w in wraps]

    def grp(g, _):
        e0 = g * LANES
        r0 = g * (LANES // 2)
        for d0 in range(0, EMBED_DIM, LANES):
            vs = [plsc.load_gather(inbuf, [d0 + iota, e0 + wraps[sh]])
                  for sh in range(LANES)]
            for sh in range(LANES):
                plsc.store_scatter(
                    outbuf, [r0 + rows[sh], d0 + cols[sh]], vs[sh])
        return 0

    lax.fori_loop(0, ngroups, grp, 0)


def _fmt_kernel(et_hbm, out_hbm, in0, in1, in2, in3, out0, out1, out2, out3,
                sem_i0, sem_i1, sem_i2, sem_i3,
                sem_o0, sem_o1, sem_o2, sem_o3):
    wid = lax.axis_index("s") * NUM_CORES + lax.axis_index("c")
    start = wid * BIG_PER_W
    ins = ((in0, sem_i0), (in1, sem_i1), (in2, sem_i2), (in3, sem_i3))
    outs = ((out0, sem_o0), (out1, sem_o1), (out2, sem_o2), (out3, sem_o3))

    def fire_in(b, buf, sem):
        lane = pl.multiple_of(b * BIGLANES, 128)
        for dblk in range(EMBED_DIM // 8):
            cp = pltpu.async_copy(
                et_hbm.at[pl.ds(dblk * 8, 8), pl.ds(lane, BIGLANES)],
                buf.at[pl.ds(dblk * 8, 8), :], sem)
        return cp

    def fire_out(b, buf, sem):
        row = pl.multiple_of(b * BIGROWS, 8)
        return pltpu.async_copy(buf, out_hbm.at[pl.ds(row, BIGROWS), :], sem)

    def wait_in(buf, sem):
        for dblk in range(EMBED_DIM // 8):
            pltpu.make_async_copy(
                et_hbm.at[pl.ds(0, 8), pl.ds(0, BIGLANES)],
                buf.at[pl.ds(dblk * 8, 8), :], sem).wait()

    def wait_out(buf, sem):
        pltpu.make_async_copy(
            buf, out_hbm.at[pl.ds(0, BIGROWS), :], sem).wait()

    for sl in range(NSLOT):
        fire_in(start + sl, ins[sl][0], ins[sl][1])

    def body(k, _):
        b = start + NSLOT * k
        for sl in range(NSLOT):
            bb = b + sl
            ibuf, isem = ins[sl]
            obuf, osem = outs[sl]
            wait_in(ibuf, isem)

            @pl.when(k > 0)
            def _():
                wait_out(obuf, osem)

            _transpose_block(ibuf, obuf, BIGLANES // LANES)
            fire_out(bb, obuf, osem)
            fire_in(bb + NSLOT, ibuf, isem)
        return 0

    lax.fori_loop(0, BIG_PER_W // NSLOT, body, 0)

    for sl in range(NSLOT):
        wait_in(ins[sl][0], ins[sl][1])
        wait_out(outs[sl][0], outs[sl][1])

    @pl.when(wid < NREM)
    def _():
        bb = NW * BIG_PER_W + wid
        fire_in(bb, in0, sem_i0)
        wait_in(in0, sem_i0)
        _transpose_block(in0, out0, BIGLANES // LANES)
        fire_out(bb, out0, sem_o0).wait()

    @pl.when(wid == NREM)
    def _():
        lane = pl.multiple_of(NBIG * BIGLANES, 128)
        pltpu.async_copy(
            et_hbm.at[:, pl.ds(lane, 128)],
            in0.at[:, pl.ds(0, 128)], sem_i0).wait()
        _transpose_block(in0, out0, 128 // LANES)
        pltpu.async_copy(
            out0.at[pl.ds(0, 64), :],
            out_hbm.at[pl.ds(pl.multiple_of(NBIG * BIGROWS, 8), 64), :],
            sem_o0).wait()


def _score_kernel(s_hbm, p_hbm, o_hbm, e2_hbm, r2_hbm, out_hbm,
                  s2, p2, o2, hs, hp, ho,
                  sbuf, pbuf, obuf, stage, out_v,
                  sem_s, sem_p, sem_o):
    wid = lax.axis_index("s") * NUM_CORES + lax.axis_index("c")
    base = wid * B_PER_W

    pltpu.sync_copy(s_hbm.at[pl.ds(base, B_PER_W)], s2)
    pltpu.sync_copy(p_hbm.at[pl.ds(base, B_PER_W)], p2)
    pltpu.sync_copy(o_hbm.at[pl.ds(base, B_PER_W)], o2)
    for g in range(B_PER_W // LANES):
        sl = pl.ds(g * LANES, LANES)
        for idx_ref, h_ref in ((s2, hs), (p2, hp), (o2, ho)):
            v = idx_ref[sl]
            h_ref[sl] = (v & 1) * EMBED_DIM
            idx_ref[sl] = v >> 1

    def chunk_body(c, _):
        cb = c * CHUNK
        cp_s = pltpu.async_copy(e2_hbm.at[s2.at[pl.ds(cb, CHUNK)]], sbuf, sem_s)
        cp_p = pltpu.async_copy(r2_hbm.at[p2.at[pl.ds(cb, CHUNK)]], pbuf, sem_p)
        cp_o = pltpu.async_copy(e2_hbm.at[o2.at[pl.ds(cb, CHUNK)]], obuf, sem_o)
        cp_s.wait()
        cp_p.wait()
        cp_o.wait()

        for g in range(GROUPS):
            gsl = pl.ds(cb + g * LANES, LANES)
            hsv = hs[gsl]
            hpv = hp[gsl]
            hov = ho[gsl]
            for j in range(LANES):
                row = g * LANES + j
                a = hsv[j]
                b = hpv[j]
                cofs = hov[j]
                acc = (sbuf[row, pl.ds(a, LANES)]
                       * pbuf[row, pl.ds(b, LANES)]
                       * obuf[row, pl.ds(cofs, LANES)])
                for d in range(1, EMBED_DIM // LANES):
                    acc = acc + (sbuf[row, pl.ds(a + d * LANES, LANES)]
                                 * pbuf[row, pl.ds(b + d * LANES, LANES)]
                                 * obuf[row, pl.ds(cofs + d * LANES, LANES)])
                stage[pl.ds(j * LANES, LANES)] = acc
            col = lax.iota(jnp.int32, LANES) * LANES
            out_vec = plsc.load_gather(stage, [col])
            for l in range(1, LANES):
                out_vec = out_vec + plsc.load_gather(stage, [col + l])
            out_v[gsl] = out_vec
        return 0

    lax.fori_loop(0, NCHUNK, chunk_body, 0)

    pltpu.sync_copy(out_v, out_hbm.at[pl.ds(base, B_PER_W)])


_SC_PARAMS = pltpu.CompilerParams(
    needs_layout_passes=False, use_tc_tiling_on_sc=True)


@jax.jit
def _run(s, p, o, entity_embeddings, relation_embeddings):
    mesh = plsc.VectorSubcoreMesh(core_axis_name="c", subcore_axis_name="s")
    et = entity_embeddings.T
    fmt = functools.partial(
        pl.kernel,
        out_type=jax.ShapeDtypeStruct((SCRATCH_ROWS, D2), jnp.float32),
        mesh=mesh,
        compiler_params=_SC_PARAMS,
        scratch_types=(
            [pltpu.VMEM((EMBED_DIM, BIGLANES), jnp.float32)] * NSLOT
            + [pltpu.VMEM((BIGROWS, D2), jnp.float32)] * NSLOT
            + [pltpu.SemaphoreType.DMA] * (2 * NSLOT)
        ),
    )(_fmt_kernel)
    e2 = fmt(et)

    r2 = relation_embeddings.reshape(
        relation_embeddings.shape[0] // 2, D2)
    score = functools.partial(
        pl.kernel,
        out_type=jax.ShapeDtypeStruct((BATCH,), jnp.float32),
        mesh=mesh,
        compiler_params=_SC_PARAMS,
        scratch_types=[
            pltpu.VMEM((B_PER_W,), jnp.int32),
            pltpu.VMEM((B_PER_W,), jnp.int32),
            pltpu.VMEM((B_PER_W,), jnp.int32),
            pltpu.VMEM((B_PER_W,), jnp.int32),
            pltpu.VMEM((B_PER_W,), jnp.int32),
            pltpu.VMEM((B_PER_W,), jnp.int32),
            pltpu.VMEM((CHUNK, D2), jnp.float32),
            pltpu.VMEM((CHUNK, D2), jnp.float32),
            pltpu.VMEM((CHUNK, D2), jnp.float32),
            pltpu.VMEM((LANES * LANES,), jnp.float32),
            pltpu.VMEM((B_PER_W,), jnp.float32),
            pltpu.SemaphoreType.DMA,
            pltpu.SemaphoreType.DMA,
            pltpu.SemaphoreType.DMA,
        ],
    )(_score_kernel)
    return score(s, p, o, e2, r2)


def kernel(s, p, o, entity_embeddings, relation_embeddings):
    return _run(s.astype(jnp.int32), p.astype(jnp.int32), o.astype(jnp.int32),
                entity_embeddings, relation_embeddings)
